# baseline (speedup 1.0000x reference)
import jax
import jax.numpy as jnp
from jax.experimental import pallas as pl
from jax.experimental.pallas import tpu as pltpu

_BLK = 14


def _copy_body(in_ref, out_ref):
    out_ref[...] = in_ref[...]


def kernel(x):
    B, T, D = x.shape
    xt = jax.lax.transpose(x, (1, 0, 2))
    yt = pl.pallas_call(
        _copy_body,
        out_shape=jax.ShapeDtypeStruct((T, B, D), x.dtype),
        grid=(T // _BLK,),
        in_specs=[pl.BlockSpec((_BLK, B, D), lambda i: (i, 0, 0))],
        out_specs=pl.BlockSpec((_BLK, B, D), lambda i: (i, 0, 0)),
        compiler_params=pltpu.CompilerParams(
            dimension_semantics=("parallel",),
        ),
    )(xt)
    return jax.lax.transpose(yt, (1, 0, 2))


# speedup vs baseline: 1.1151x; 1.1151x over previous
import jax
import jax.numpy as jnp
from jax.experimental import pallas as pl
from jax.experimental.pallas import tpu as pltpu

_BLK = 49


def _copy_body(in_ref, out_ref):
    out_ref[...] = in_ref[...]


def kernel(x):
    B, T, D = x.shape
    xt = jax.lax.transpose(x, (1, 0, 2))
    yt = pl.pallas_call(
        _copy_body,
        out_shape=jax.ShapeDtypeStruct((T, B, D), x.dtype),
        grid=(T // _BLK,),
        in_specs=[pl.BlockSpec((_BLK, B, D), lambda i: (i, 0, 0))],
        out_specs=pl.BlockSpec((_BLK, B, D), lambda i: (i, 0, 0)),
        compiler_params=pltpu.CompilerParams(
            dimension_semantics=("parallel",),
        ),
    )(xt)
    return jax.lax.transpose(yt, (1, 0, 2))
